# baseline (speedup 1.0000x reference)
import functools

import jax
import jax.numpy as jnp
from jax import lax
from jax.experimental import pallas as pl
from jax.experimental.pallas import tpu as pltpu
from jax.experimental.pallas import tpu_sc as plsc

CENTER_R = 16.0
NROWS = 33
NPAD = 48
NGATH = 40
HW = 128
L = 16

NC = 2
NS = 16
NW = NC * NS


def _smooth_l1_vec(d):
    ad = jnp.abs(d)
    return jnp.where(ad < 1.0, 0.5 * d * d, ad - 0.5)


def _make_sc_call(B, H, W):
    batches_per_w = B // NW

    mesh = plsc.VectorSubcoreMesh(core_axis_name="c", subcore_axis_name="s")

    @functools.partial(
        pl.kernel,
        out_type=jax.ShapeDtypeStruct((NW, 2, L), jnp.float32),
        mesh=mesh,
        scratch_types=[
            pltpu.VMEM((2, B), jnp.float32),
            pltpu.VMEM((2 * NPAD,), jnp.int32),
            pltpu.VMEM((4 * NPAD,), jnp.int32),
            pltpu.VMEM((2 * NPAD,), jnp.int32),
            pltpu.VMEM((4 * NPAD,), jnp.int32),
            pltpu.VMEM((2 * NGATH, HW), jnp.float32),
            pltpu.VMEM((4 * NGATH, HW), jnp.float32),
            pltpu.VMEM((2 * NGATH, HW), jnp.float32),
            pltpu.VMEM((4 * NGATH, HW), jnp.float32),
            pltpu.VMEM((2, L), jnp.float32),
            pltpu.SemaphoreType.DMA,
        ],
        compiler_params=pltpu.CompilerParams(needs_layout_passes=False),
    )
    def sc_loss(cls_hbm, loc_hbm, cr_hbm, out_hbm,
                cr_v, idx_c0, idx_l0, idx_c1, idx_l1,
                buf_c0, buf_l0, buf_c1, buf_l1, res_v, sem):
        wid = lax.axis_index("s") * NC + lax.axis_index("c")
        pltpu.sync_copy(cr_hbm, cr_v)

        ii = lax.iota(jnp.int32, L)
        iif = ii.astype(jnp.float32)

        def batch_params(b):
            q = b // L
            lane = b - q * L
            chunk0 = cr_v[0, pl.ds(q * L, L)]
            chunk1 = cr_v[1, pl.ds(q * L, L)]
            sel = (ii == lane).astype(jnp.float32)
            c0 = jnp.sum(chunk0 * sel) * jnp.float32(H - 1)
            c1 = jnp.sum(chunk1 * sel) * jnp.float32(W - 1)
            r0 = jnp.maximum(c0 - CENTER_R, 0.0).astype(jnp.int32)
            o = jnp.maximum(c1 - CENTER_R, 0.0).astype(jnp.int32)
            ksel = o >> 7
            woff = o - ksel * HW
            a = jnp.minimum((woff >> 4) << 4, 112 - 32 * ksel)
            straddle = (ksel == 0) & (woff > 95)
            return c0, c1, r0, ksel, a, straddle

        def build_idx(b, r0, ksel, idx_c, idx_l):
            for v in range(NROWS // L + 1):
                r = jnp.minimum(r0 + ii + v * L, H - 1)
                hc = 2 * (b * H + r) + ksel
                hl0 = 2 * ((b * 2) * H + r) + ksel
                hl1 = 2 * ((b * 2 + 1) * H + r) + ksel
                idx_c[pl.ds(v * L, L)] = hc
                idx_c[pl.ds(NPAD + v * L, L)] = hc + 1
                idx_l[pl.ds(v * L, L)] = hl0
                idx_l[pl.ds(NPAD + v * L, L)] = hl0 + 1
                idx_l[pl.ds(2 * NPAD + v * L, L)] = hl1
                idx_l[pl.ds(3 * NPAD + v * L, L)] = hl1 + 1

        def fire(idx_c, idx_l, buf_c, buf_l, straddle):
            pltpu.async_copy(
                cls_hbm.at[idx_c.at[pl.ds(0, NGATH)]],
                buf_c.at[pl.ds(0, NGATH)], sem)
            pltpu.async_copy(
                loc_hbm.at[idx_l.at[pl.ds(0, NGATH)]],
                buf_l.at[pl.ds(0, NGATH)], sem)
            pltpu.async_copy(
                loc_hbm.at[idx_l.at[pl.ds(2 * NPAD, NGATH)]],
                buf_l.at[pl.ds(2 * NGATH, NGATH)], sem)

            pltpu.async_copy(
                cls_hbm.at[idx_c.at[pl.ds(NPAD, NGATH)]],
                buf_c.at[pl.ds(NGATH, NGATH)], sem)
            pltpu.async_copy(
                loc_hbm.at[idx_l.at[pl.ds(NPAD, NGATH)]],
                buf_l.at[pl.ds(NGATH, NGATH)], sem)
            pltpu.async_copy(
                loc_hbm.at[idx_l.at[pl.ds(3 * NPAD, NGATH)]],
                buf_l.at[pl.ds(3 * NGATH, NGATH)], sem)

        def drain(idx_c, idx_l, buf_c, buf_l, straddle):
            pltpu.make_async_copy(
                cls_hbm.at[idx_c.at[pl.ds(0, NGATH)]],
                buf_c.at[pl.ds(0, NGATH)], sem).wait()
            pltpu.make_async_copy(
                loc_hbm.at[idx_l.at[pl.ds(0, NGATH)]],
                buf_l.at[pl.ds(0, NGATH)], sem).wait()
            pltpu.make_async_copy(
                loc_hbm.at[idx_l.at[pl.ds(2 * NPAD, NGATH)]],
                buf_l.at[pl.ds(2 * NGATH, NGATH)], sem).wait()

            pltpu.make_async_copy(
                cls_hbm.at[idx_c.at[pl.ds(NPAD, NGATH)]],
                buf_c.at[pl.ds(NGATH, NGATH)], sem).wait()
            pltpu.make_async_copy(
                loc_hbm.at[idx_l.at[pl.ds(NPAD, NGATH)]],
                buf_l.at[pl.ds(NGATH, NGATH)], sem).wait()
            pltpu.make_async_copy(
                loc_hbm.at[idx_l.at[pl.ds(3 * NPAD, NGATH)]],
                buf_l.at[pl.ds(3 * NGATH, NGATH)], sem).wait()

        idx_cs = (idx_c0, idx_c1)
        idx_ls = (idx_l0, idx_l1)
        buf_cs = (buf_c0, buf_c1)
        buf_ls = (buf_l0, buf_l1)

        params = []
        for t in range(batches_per_w):
            b = wid * batches_per_w + t
            c0, c1, r0, ksel, a, straddle = batch_params(b)
            build_idx(b, r0, ksel, idx_cs[t], idx_ls[t])
            fire(idx_cs[t], idx_ls[t], buf_cs[t], buf_ls[t], straddle)
            params.append((c0, c1, r0, ksel, a, straddle))

        total = jnp.zeros((L,), jnp.float32)
        count = jnp.zeros((L,), jnp.float32)
        for t in range(batches_per_w):
            c0, c1, r0, ksel, a, straddle = params[t]
            buf_c = buf_cs[t]
            buf_l = buf_ls[t]
            drain(idx_cs[t], idx_ls[t], buf_c, buf_l, straddle)

            cols = []
            for h in range(3):
                p = a + h * L
                j = (ksel * HW + p).astype(jnp.float32) + iif
                bias1 = c1 - j
                cols.append((p >> 7, p & (HW - 1), bias1,
                             jnp.abs(bias1) <= CENTER_R))
            r0f = r0.astype(jnp.float32)

            def row_body(r, carry):
                tot, cnt = carry
                rf = r0f + r.astype(jnp.float32)
                rmask = (jnp.abs(c0 - rf) <= CENTER_R) & (rf <= jnp.float32(H - 1))
                bias0 = c0 - rf
                for h in range(3):
                    hsel, off, bias1, cmask = cols[h]
                    rh = r + NGATH * hsel
                    clsv = buf_c[rh, pl.ds(off, L)]
                    loc0 = buf_l[rh, pl.ds(off, L)]
                    loc1 = buf_l[2 * NGATH + rh, pl.ds(off, L)]
                    m = cmask & rmask & (clsv > 0.0)
                    val = _smooth_l1_vec(loc0 - bias0) + _smooth_l1_vec(loc1 - bias1)
                    tot = tot + jnp.where(m, val, 0.0)
                    cnt = cnt + jnp.where(m, 1.0, 0.0)
                return tot, cnt

            total, count = lax.fori_loop(0, NROWS, row_body, (total, count))

        res_v[0, pl.ds(0, L)] = total
        res_v[1, pl.ds(0, L)] = count
        pltpu.sync_copy(res_v, out_hbm.at[wid])

    return sc_loss


def kernel(cls_input, loc_input, center_rate):
    B, _, H, W = cls_input.shape
    cls_tab = cls_input.reshape(B * H * 2, HW)
    loc_tab = loc_input.reshape(B * 2 * H * 2, HW)
    partials = _make_sc_call(B, H, W)(cls_tab, loc_tab, center_rate)
    total = partials[:, 0, :].sum()
    count = partials[:, 1, :].sum()
    loss = total / jnp.maximum(count * 2.0, 1.0)
    return jnp.where(count == 0, 0.0, loss)


# speedup vs baseline: 1.0004x; 1.0004x over previous
import functools

import jax
import jax.numpy as jnp
from jax import lax
from jax.experimental import pallas as pl
from jax.experimental.pallas import tpu as pltpu
from jax.experimental.pallas import tpu_sc as plsc

CENTER_R = 16.0
NROWS = 33
NPAD = 48
NGATH = 40
HW = 128
L = 16

NC = 2
NS = 16
NW = NC * NS


def _smooth_l1_vec(d):
    ad = jnp.abs(d)
    return jnp.where(ad < 1.0, 0.5 * d * d, ad - 0.5)


def _make_sc_call(B, H, W):
    batches_per_w = B // NW

    mesh = plsc.VectorSubcoreMesh(core_axis_name="c", subcore_axis_name="s")

    @functools.partial(
        pl.kernel,
        out_type=jax.ShapeDtypeStruct((NW, 2, L), jnp.float32),
        mesh=mesh,
        scratch_types=[
            pltpu.VMEM((2, B), jnp.float32),
            pltpu.VMEM((2 * NPAD,), jnp.int32),
            pltpu.VMEM((4 * NPAD,), jnp.int32),
            pltpu.VMEM((2 * NPAD,), jnp.int32),
            pltpu.VMEM((4 * NPAD,), jnp.int32),
            pltpu.VMEM((2 * NGATH, HW), jnp.float32),
            pltpu.VMEM((4 * NGATH, HW), jnp.float32),
            pltpu.VMEM((2 * NGATH, HW), jnp.float32),
            pltpu.VMEM((4 * NGATH, HW), jnp.float32),
            pltpu.VMEM((2, L), jnp.float32),
            pltpu.SemaphoreType.DMA,
        ],
        compiler_params=pltpu.CompilerParams(needs_layout_passes=False),
    )
    def sc_loss(cls_hbm, loc_hbm, cr_hbm, out_hbm,
                cr_v, idx_c0, idx_l0, idx_c1, idx_l1,
                buf_c0, buf_l0, buf_c1, buf_l1, res_v, sem):
        wid = lax.axis_index("s") * NC + lax.axis_index("c")
        pltpu.sync_copy(cr_hbm, cr_v)

        ii = lax.iota(jnp.int32, L)
        iif = ii.astype(jnp.float32)

        def batch_params(b):
            q = b // L
            lane = b - q * L
            chunk0 = cr_v[0, pl.ds(q * L, L)]
            chunk1 = cr_v[1, pl.ds(q * L, L)]
            sel = (ii == lane).astype(jnp.float32)
            c0 = jnp.sum(chunk0 * sel) * jnp.float32(H - 1)
            c1 = jnp.sum(chunk1 * sel) * jnp.float32(W - 1)
            r0 = jnp.maximum(c0 - CENTER_R, 0.0).astype(jnp.int32)
            o = jnp.maximum(c1 - CENTER_R, 0.0).astype(jnp.int32)
            ksel = o >> 7
            woff = o - ksel * HW
            a = jnp.minimum((woff >> 4) << 4, 112 - 32 * ksel)
            straddle = (ksel == 0) & (woff > 95)
            return c0, c1, r0, ksel, a, straddle

        def build_idx(b, r0, ksel, idx_c, idx_l):
            for v in range(NROWS // L + 1):
                r = jnp.minimum(r0 + ii + v * L, H - 1)
                hc = 2 * (b * H + r) + ksel
                hl0 = 2 * ((b * 2) * H + r) + ksel
                hl1 = 2 * ((b * 2 + 1) * H + r) + ksel
                idx_c[pl.ds(v * L, L)] = hc
                idx_c[pl.ds(NPAD + v * L, L)] = hc + 1
                idx_l[pl.ds(v * L, L)] = hl0
                idx_l[pl.ds(NPAD + v * L, L)] = hl0 + 1
                idx_l[pl.ds(2 * NPAD + v * L, L)] = hl1
                idx_l[pl.ds(3 * NPAD + v * L, L)] = hl1 + 1

        def fire(idx_c, idx_l, buf_c, buf_l, straddle):
            cps = []
            cps.append(pltpu.async_copy(
                cls_hbm.at[idx_c.at[pl.ds(0, NGATH)]],
                buf_c.at[pl.ds(0, NGATH)], sem))
            cps.append(pltpu.async_copy(
                loc_hbm.at[idx_l.at[pl.ds(0, NGATH)]],
                buf_l.at[pl.ds(0, NGATH)], sem))
            cps.append(pltpu.async_copy(
                loc_hbm.at[idx_l.at[pl.ds(2 * NPAD, NGATH)]],
                buf_l.at[pl.ds(2 * NGATH, NGATH)], sem))
            cps.append(pltpu.async_copy(
                cls_hbm.at[idx_c.at[pl.ds(NPAD, NGATH)]],
                buf_c.at[pl.ds(NGATH, NGATH)], sem))
            cps.append(pltpu.async_copy(
                loc_hbm.at[idx_l.at[pl.ds(NPAD, NGATH)]],
                buf_l.at[pl.ds(NGATH, NGATH)], sem))
            cps.append(pltpu.async_copy(
                loc_hbm.at[idx_l.at[pl.ds(3 * NPAD, NGATH)]],
                buf_l.at[pl.ds(3 * NGATH, NGATH)], sem))
            return cps

        def drain(cps):
            for cp in cps:
                cp.wait()

        idx_cs = (idx_c0, idx_c1)
        idx_ls = (idx_l0, idx_l1)
        buf_cs = (buf_c0, buf_c1)
        buf_ls = (buf_l0, buf_l1)

        params = []
        for t in range(batches_per_w):
            b = wid * batches_per_w + t
            c0, c1, r0, ksel, a, straddle = batch_params(b)
            build_idx(b, r0, ksel, idx_cs[t], idx_ls[t])
            cps = fire(idx_cs[t], idx_ls[t], buf_cs[t], buf_ls[t], straddle)
            params.append((c0, c1, r0, ksel, a, straddle, cps))

        total = jnp.zeros((L,), jnp.float32)
        count = jnp.zeros((L,), jnp.float32)
        for t in range(batches_per_w):
            c0, c1, r0, ksel, a, straddle, cps = params[t]
            buf_c = buf_cs[t]
            buf_l = buf_ls[t]
            drain(cps)

            cols = []
            for h in range(3):
                p = a + h * L
                j = (ksel * HW + p).astype(jnp.float32) + iif
                bias1 = c1 - j
                cols.append((p >> 7, p & (HW - 1), bias1,
                             jnp.abs(bias1) <= CENTER_R))
            r0f = r0.astype(jnp.float32)

            def row_body(r, carry):
                tot, cnt = carry
                rf = r0f + r.astype(jnp.float32)
                rmask = (jnp.abs(c0 - rf) <= CENTER_R) & (rf <= jnp.float32(H - 1))
                bias0 = c0 - rf
                for h in range(3):
                    hsel, off, bias1, cmask = cols[h]
                    rh = r + NGATH * hsel
                    clsv = buf_c[rh, pl.ds(off, L)]
                    loc0 = buf_l[rh, pl.ds(off, L)]
                    loc1 = buf_l[2 * NGATH + rh, pl.ds(off, L)]
                    m = cmask & rmask & (clsv > 0.0)
                    val = _smooth_l1_vec(loc0 - bias0) + _smooth_l1_vec(loc1 - bias1)
                    tot = tot + jnp.where(m, val, 0.0)
                    cnt = cnt + jnp.where(m, 1.0, 0.0)
                return tot, cnt

            total, count = lax.fori_loop(0, NROWS, row_body, (total, count))

        res_v[0, pl.ds(0, L)] = total
        res_v[1, pl.ds(0, L)] = count
        pltpu.sync_copy(res_v, out_hbm.at[wid])

    return sc_loss


def kernel(cls_input, loc_input, center_rate):
    B, _, H, W = cls_input.shape
    cls_tab = cls_input.reshape(B * H * 2, HW)
    loc_tab = loc_input.reshape(B * 2 * H * 2, HW)
    partials = _make_sc_call(B, H, W)(cls_tab, loc_tab, center_rate)
    total = partials[:, 0, :].sum()
    count = partials[:, 1, :].sum()
    loss = total / jnp.maximum(count * 2.0, 1.0)
    return jnp.where(count == 0, 0.0, loss)


# speedup vs baseline: 2.6110x; 2.6100x over previous
import functools

import jax
import jax.numpy as jnp
from jax import lax
from jax.experimental import pallas as pl
from jax.experimental.pallas import tpu as pltpu
from jax.experimental.pallas import tpu_sc as plsc

CENTER_R = 16.0
NROWS = 33
NPAD = 48
NGATH = 40
L = 16

NC = 2
NS = 16
NW = NC * NS


def _smooth_l1_vec(d):
    ad = jnp.abs(d)
    return jnp.where(ad < 1.0, 0.5 * d * d, ad - 0.5)


def _make_sc_call(B, H, W):
    batches_per_w = B // NW

    mesh = plsc.VectorSubcoreMesh(core_axis_name="c", subcore_axis_name="s")

    @functools.partial(
        pl.kernel,
        out_type=jax.ShapeDtypeStruct((NW, 2, L), jnp.float32),
        mesh=mesh,
        scratch_types=[
            pltpu.VMEM((2, B), jnp.float32),
            pltpu.VMEM((NPAD,), jnp.int32),
            pltpu.VMEM((2 * NPAD,), jnp.int32),
            pltpu.VMEM((NPAD,), jnp.int32),
            pltpu.VMEM((2 * NPAD,), jnp.int32),
            pltpu.VMEM((NGATH, 256), jnp.float32),
            pltpu.VMEM((2 * NGATH, 256), jnp.float32),
            pltpu.VMEM((NGATH, 256), jnp.float32),
            pltpu.VMEM((2 * NGATH, 256), jnp.float32),
            pltpu.VMEM((2, L), jnp.float32),
            pltpu.SemaphoreType.DMA,
        ],
        compiler_params=pltpu.CompilerParams(needs_layout_passes=False),
    )
    def sc_loss(cls_hbm, loc_hbm, cr_hbm, out_hbm,
                cr_v, idx_c0, idx_l0, idx_c1, idx_l1,
                buf_c0, buf_l0, buf_c1, buf_l1, res_v, sem):
        wid = lax.axis_index("s") * NC + lax.axis_index("c")
        pltpu.sync_copy(cr_hbm, cr_v)

        ii = lax.iota(jnp.int32, L)
        iif = ii.astype(jnp.float32)

        def batch_params(b):
            q = b // L
            lane = b - q * L
            chunk0 = cr_v[0, pl.ds(q * L, L)]
            chunk1 = cr_v[1, pl.ds(q * L, L)]
            sel = (ii == lane).astype(jnp.float32)
            c0 = jnp.sum(chunk0 * sel) * jnp.float32(H - 1)
            c1 = jnp.sum(chunk1 * sel) * jnp.float32(W - 1)
            r0 = jnp.maximum(c0 - CENTER_R, 0.0).astype(jnp.int32)
            o = jnp.maximum(c1 - CENTER_R, 0.0).astype(jnp.int32)
            a = jnp.minimum((o >> 4) << 4, W - 3 * L)
            return c0, c1, r0, a

        def build_idx(b, r0, idx_c, idx_l):
            for v in range(NPAD // L):
                r = jnp.minimum(r0 + ii + v * L, H - 1)
                idx_c[pl.ds(v * L, L)] = b * H + r
                idx_l[pl.ds(v * L, L)] = (b * 2) * H + r
                idx_l[pl.ds(NPAD + v * L, L)] = (b * 2 + 1) * H + r

        params = []
        copies = []
        for t in range(batches_per_w):
            b = wid * batches_per_w + t
            c0, c1, r0, a = batch_params(b)
            idx_c = (idx_c0, idx_c1)[t]
            idx_l = (idx_l0, idx_l1)[t]
            build_idx(b, r0, idx_c, idx_l)
            buf_c = (buf_c0, buf_c1)[t]
            buf_l = (buf_l0, buf_l1)[t]
            cp_c = pltpu.async_copy(
                cls_hbm.at[idx_c.at[pl.ds(0, NGATH)]], buf_c, sem)
            cp_l0 = pltpu.async_copy(
                loc_hbm.at[idx_l.at[pl.ds(0, NGATH)]],
                buf_l.at[pl.ds(0, NGATH)], sem)
            cp_l1 = pltpu.async_copy(
                loc_hbm.at[idx_l.at[pl.ds(NPAD, NGATH)]],
                buf_l.at[pl.ds(NGATH, NGATH)], sem)
            params.append((c0, c1, r0, a))
            copies.append((cp_c, cp_l0, cp_l1))

        total = jnp.zeros((L,), jnp.float32)
        count = jnp.zeros((L,), jnp.float32)
        for t in range(batches_per_w):
            c0, c1, r0, a = params[t]
            buf_c = (buf_c0, buf_c1)[t]
            buf_l = (buf_l0, buf_l1)[t]
            copies[t][0].wait()
            copies[t][1].wait()
            copies[t][2].wait()

            cols = []
            for h in range(3):
                o16 = a + h * L
                j = o16.astype(jnp.float32) + iif
                bias1 = c1 - j
                cols.append((o16, bias1, jnp.abs(bias1) <= CENTER_R))
            r0f = r0.astype(jnp.float32)

            def row_body(r, carry):
                rf = r0f + r.astype(jnp.float32)
                rmask = (jnp.abs(c0 - rf) <= CENTER_R) & (rf <= jnp.float32(H - 1))
                bias0 = c0 - rf
                out = []
                for h in range(3):
                    o16, bias1, cmask = cols[h]
                    tot, cnt = carry[h]
                    clsv = buf_c[r, pl.ds(o16, L)]
                    loc0 = buf_l[r, pl.ds(o16, L)]
                    loc1 = buf_l[NGATH + r, pl.ds(o16, L)]
                    m = cmask & rmask & (clsv > 0.0)
                    val = _smooth_l1_vec(loc0 - bias0) + _smooth_l1_vec(loc1 - bias1)
                    out.append((tot + jnp.where(m, val, 0.0),
                                cnt + jnp.where(m, 1.0, 0.0)))
                return tuple(out)

            acc = lax.fori_loop(
                0, NROWS, row_body,
                ((total, count), (jnp.zeros((L,), jnp.float32),) * 2,
                 (jnp.zeros((L,), jnp.float32),) * 2))
            total = acc[0][0] + acc[1][0] + acc[2][0]
            count = acc[0][1] + acc[1][1] + acc[2][1]

        res_v[0, pl.ds(0, L)] = total
        res_v[1, pl.ds(0, L)] = count
        pltpu.sync_copy(res_v, out_hbm.at[wid])

    return sc_loss


def kernel(cls_input, loc_input, center_rate):
    B, _, H, W = cls_input.shape
    cls_tab = cls_input.reshape(B * H, W)
    loc_tab = loc_input.reshape(B * 2 * H, W)
    partials = _make_sc_call(B, H, W)(cls_tab, loc_tab, center_rate)
    total = partials[:, 0, :].sum()
    count = partials[:, 1, :].sum()
    loss = total / jnp.maximum(count * 2.0, 1.0)
    return jnp.where(count == 0, 0.0, loss)
